# Initial kernel scaffold; baseline (speedup 1.0000x reference)
import jax
import jax.numpy as jnp
from jax.experimental import pallas as pl


def kernel(cate_0, cate_1, cate_2, target, mask, interaction, cont_0, gather_index, emb_interaction, emb_cate_0, emb_cate_1, emb_cate_2, W_cont, b_cont, ln_gamma, ln_beta):
    raise NotImplementedError("write your pallas kernel here")


# speedup vs baseline: 4.0127x; 4.0127x over previous
import functools

import jax
import jax.numpy as jnp
from jax import lax
from jax.experimental import pallas as pl
from jax.experimental.pallas import tpu as pltpu
from jax.experimental.pallas import tpu_sc as plsc

B = 1024
L = 200
D = 32
BL = B * L

_NC = 2
_NS = 16
_NW = _NC * _NS

_PER_W = BL // _NW
_CHUNK = 128
_NCHUNK = _PER_W // _CHUNK


def _sc_gather3(t0, t1, t2, idx_all):
    mesh = plsc.VectorSubcoreMesh(core_axis_name="c", subcore_axis_name="s")

    @functools.partial(
        pl.kernel,
        out_type=[jax.ShapeDtypeStruct((BL, D), jnp.float32)] * 3,
        mesh=mesh,
        scratch_types=[
            pltpu.VMEM((_NCHUNK, _CHUNK), jnp.int32),
            pltpu.VMEM((_CHUNK, D), jnp.float32),
            pltpu.SemaphoreType.DMA,
        ],
        compiler_params=pltpu.CompilerParams(use_tc_tiling_on_sc=False),
    )
    def k(t0h, t1h, t2h, idxh, o0h, o1h, o2h, idx_v, rows_v, sem):
        wid = lax.axis_index("s") * _NC + lax.axis_index("c")
        tabs = (t0h, t1h, t2h)
        outs = (o0h, o1h, o2h)
        for t in range(3):
            pltpu.sync_copy(idxh.at[t * _NW + wid], idx_v)
            tab = tabs[t]
            outh = outs[t]

            def body(j, carry):
                pltpu.async_copy(tab.at[idx_v.at[j]], rows_v, sem).wait()
                pltpu.sync_copy(
                    rows_v, outh.at[pl.ds(wid * _PER_W + j * _CHUNK, _CHUNK)]
                )
                return carry

            lax.fori_loop(0, _NCHUNK, body, 0)

    return k(t0, t1, t2, idx_all)


_TC_ROWS = 2048


def _tc_body(inter_ref, cont_ref, emb_ref, w_ref, b_ref, g_ref, beta_ref,
             out_i_ref, out_c_ref):
    iv = inter_ref[...]
    c = cont_ref[...]
    rows = emb_ref[...]
    r0 = rows[0:1, :]
    r1 = rows[1:2, :]
    r2 = rows[2:3, :]
    out_i_ref[...] = jnp.where(iv == 0, r0, jnp.where(iv == 1, r1, r2))
    h = c * w_ref[...] + b_ref[...]
    mu = jnp.mean(h, axis=-1, keepdims=True)
    var = jnp.mean((h - mu) ** 2, axis=-1, keepdims=True)
    out_c_ref[...] = (h - mu) * lax.rsqrt(var + 1e-5) * g_ref[...] + beta_ref[...]


def _tc_dense(interaction, cont, emb_interaction, W_cont, b_cont, ln_gamma, ln_beta):
    grid = BL // _TC_ROWS
    return pl.pallas_call(
        _tc_body,
        grid=(grid,),
        in_specs=[
            pl.BlockSpec((_TC_ROWS, 1), lambda i: (i, 0)),
            pl.BlockSpec((_TC_ROWS, 1), lambda i: (i, 0)),
            pl.BlockSpec((3, D), lambda i: (0, 0)),
            pl.BlockSpec((1, D), lambda i: (0, 0)),
            pl.BlockSpec((1, D), lambda i: (0, 0)),
            pl.BlockSpec((1, D), lambda i: (0, 0)),
            pl.BlockSpec((1, D), lambda i: (0, 0)),
        ],
        out_specs=[
            pl.BlockSpec((_TC_ROWS, D), lambda i: (i, 0)),
            pl.BlockSpec((_TC_ROWS, D), lambda i: (i, 0)),
        ],
        out_shape=[jax.ShapeDtypeStruct((BL, D), jnp.float32)] * 2,
    )(interaction, cont, emb_interaction, W_cont, b_cont, ln_gamma, ln_beta)


def kernel(cate_0, cate_1, cate_2, target, mask, interaction, cont_0,
           gather_index, emb_interaction, emb_cate_0, emb_cate_1, emb_cate_2,
           W_cont, b_cont, ln_gamma, ln_beta):
    idx_all = jnp.concatenate([
        cate_0.reshape(_NW, _NCHUNK, _CHUNK),
        cate_1.reshape(_NW, _NCHUNK, _CHUNK),
        cate_2.reshape(_NW, _NCHUNK, _CHUNK),
    ], axis=0).astype(jnp.int32)
    e0, e1, e2 = _sc_gather3(emb_cate_0, emb_cate_1, emb_cate_2, idx_all)

    ei, ec = _tc_dense(
        interaction.reshape(BL, 1).astype(jnp.int32),
        cont_0.reshape(BL, 1),
        emb_interaction,
        W_cont.reshape(1, D),
        b_cont.reshape(1, D),
        ln_gamma.reshape(1, D),
        ln_beta.reshape(1, D),
    )

    return (
        e0.reshape(B, L, D),
        e1.reshape(B, L, D),
        e2.reshape(B, L, D),
        mask,
        ei.reshape(B, L, D),
        ec.reshape(B, L, D),
        gather_index,
    )
